# Initial kernel scaffold; baseline (speedup 1.0000x reference)
import jax
import jax.numpy as jnp
from jax.experimental import pallas as pl


def kernel(xyz, feats, anchors, inter_idx, inter_w, W):
    raise NotImplementedError("write your pallas kernel here")


# speedup vs baseline: 23.5560x; 23.5560x over previous
import functools

import jax
import jax.numpy as jnp
from jax import lax
from jax.experimental import pallas as pl
from jax.experimental.pallas import tpu as pltpu
from jax.experimental.pallas import tpu_sc as plsc

_C_IN, _N, _A, _NN, _KS, _C_OUT = 32, 10000, 12, 16, 3, 64
_D = _A * _C_IN
_M = _KS * _C_IN
_L = 16

_NC, _NS = 2, 16
_NW = _NC * _NS
_CH = 8
_NCHUNK = _N // _CH
_ITERS = -(-_NCHUNK // _NW)


def _sc_body(table_hbm, idx_hbm, w_hbm, nf_hbm, idx_v, rows_v, w_v, out_v, sem):
    wid = lax.axis_index("s") * _NC + lax.axis_index("c")

    def chunk_body(ch, carry):
        cid = ch * _NW + wid

        @pl.when(cid < _NCHUNK)
        def _():
            p0 = cid * _CH
            pltpu.sync_copy(idx_hbm.at[pl.ds(p0 * _NN, _CH * _NN)], idx_v)
            pltpu.async_copy(table_hbm.at[idx_v], rows_v, sem).wait()
            pltpu.sync_copy(w_hbm.at[pl.ds(p0, _CH)], w_v)

            def pt_body(pt, c1):
                def a_body(a, c2):
                    wv = [w_v[pt, a, k, :] for k in range(_KS)]
                    acc = [[jnp.zeros((_L,), jnp.float32) for _ in range(2)]
                           for _ in range(_KS)]
                    for n in range(_NN):
                        r0 = rows_v[pt * _NN + n, pl.ds(a * _C_IN, _L)]
                        r1 = rows_v[pt * _NN + n, pl.ds(a * _C_IN + _L, _L)]
                        for k in range(_KS):
                            s = wv[k][n]
                            acc[k][0] = acc[k][0] + s * r0
                            acc[k][1] = acc[k][1] + s * r1
                    for k in range(_KS):
                        out_v[pt, pl.ds(a * _M + k * _C_IN, _L)] = acc[k][0]
                        out_v[pt, pl.ds(a * _M + k * _C_IN + _L, _L)] = acc[k][1]
                    return c2

                lax.fori_loop(0, _A, a_body, c1)
                return c1

            lax.fori_loop(0, _CH, pt_body, 0)
            pltpu.sync_copy(out_v, nf_hbm.at[pl.ds(p0, _CH)])

        return carry

    lax.fori_loop(0, _ITERS, chunk_body, 0)


@functools.cache
def _get_sc_nf():
    return pl.kernel(
        _sc_body,
        out_type=jax.ShapeDtypeStruct((_N, _A * _M), jnp.float32),
        mesh=plsc.VectorSubcoreMesh(
            core_axis_name="c", subcore_axis_name="s",
            num_cores=_NC, num_subcores=_NS),
        scratch_types=[
            pltpu.VMEM((_CH * _NN,), jnp.int32),
            pltpu.VMEM((_CH * _NN, _D), jnp.float32),
            pltpu.VMEM((_CH, _A, _KS, _NN), jnp.float32),
            pltpu.VMEM((_CH, _A * _M), jnp.float32),
            pltpu.SemaphoreType.DMA,
        ],
    )

_RB = 1000
_G1 = (_N * _A) // _RB


def _tc_body(w2_ref, x_ref, o_ref, st_ref):
    ph = pl.program_id(0)
    i = pl.program_id(1)

    @pl.when(ph == 0)
    def _():
        y = lax.dot_general(x_ref[...], w2_ref[...], (((1,), (0,)), ((), ())),
                            preferred_element_type=jnp.float32)
        o_ref[pl.ds(i * _RB, _RB), :] = y

        @pl.when(i == 0)
        def _():
            st_ref[...] = jnp.zeros_like(st_ref)

        st_ref[0, :] += jnp.sum(y, axis=0)
        st_ref[1, :] += jnp.sum(y * y, axis=0)

    @pl.when(ph == 1)
    def _():
        cnt = float(_N * _A)
        mean = st_ref[0, :] / cnt
        var = st_ref[1, :] / cnt - mean * mean
        inv = lax.rsqrt(var + 1e-5)
        y = o_ref[pl.ds(i * _RB, _RB), :]
        o_ref[pl.ds(i * _RB, _RB), :] = jnp.maximum(
            (y - mean[None, :]) * inv[None, :], 0.0)


_tc_call = pl.pallas_call(
    _tc_body,
    grid=(2, _G1),
    in_specs=[
        pl.BlockSpec((_M, _C_OUT), lambda ph, i: (0, 0)),
        pl.BlockSpec((_RB, _M), lambda ph, i: (i * (1 - ph), 0)),
    ],
    out_specs=pl.BlockSpec((_N * _A, _C_OUT), lambda ph, i: (0, 0)),
    out_shape=jax.ShapeDtypeStruct((_N * _A, _C_OUT), jnp.float32),
    scratch_shapes=[pltpu.VMEM((2, _C_OUT), jnp.float32)],
    compiler_params=pltpu.CompilerParams(vmem_limit_bytes=100 * 1024 * 1024),
)


def kernel(xyz, feats, anchors, inter_idx, inter_w, W):
    table = feats[0].transpose(1, 2, 0).reshape(_N, _D)
    idx = inter_idx[0].reshape(_N * _NN).astype(jnp.int32)
    wgt = inter_w[0]
    nf = _get_sc_nf()(table, idx, wgt)
    x = nf.reshape(_N * _A, _M)
    w2 = W.reshape(_C_OUT, _C_IN, _KS).transpose(2, 1, 0).reshape(_M, _C_OUT)
    out = _tc_call(w2, x)
    return out.T.reshape(1, _C_OUT, _N, _A)


# speedup vs baseline: 34.4764x; 1.4636x over previous
import functools

import jax
import jax.numpy as jnp
from jax import lax
from jax.experimental import pallas as pl
from jax.experimental.pallas import tpu as pltpu
from jax.experimental.pallas import tpu_sc as plsc

_C_IN, _N, _A, _NN, _KS, _C_OUT = 32, 10000, 12, 16, 3, 64
_D = _A * _C_IN
_M = _KS * _C_IN
_MP = 128
_L = 16

_NC, _NS = 2, 16
_NW = _NC * _NS
_CH = 4
_CR = _CH * _A
_NCHUNK = _N // _CH
_ITERS = -(-_NCHUNK // _NW)


def _sc_body(table_hbm, idx_hbm, w_hbm, nf_hbm,
             idx0, idx1, rows0, rows1, w0, w1, out0, out1,
             g0, g1, o0, o1, ic0, ic1, ws0, ws1):
    wid = lax.axis_index("s") * _NC + lax.axis_index("c")
    n_me = (_NCHUNK - wid + _NW - 1) // _NW

    def start_idx(c, idxv, icsem):
        pltpu.async_copy(idx_hbm.at[pl.ds(c * _CH * _NN, _CH * _NN)], idxv,
                         icsem)

    def wait_idx(idxv, icsem):
        pltpu.make_async_copy(idx_hbm.at[pl.ds(0, _CH * _NN)], idxv,
                              icsem).wait()

    def start_w(c, wv, wsem):
        pltpu.async_copy(w_hbm.at[pl.ds(c * _CH, _CH)], wv, wsem)

    def wait_w(wv, wsem):
        pltpu.make_async_copy(w_hbm.at[pl.ds(0, _CH)], wv, wsem).wait()

    def start_gather(idxv, rowsv, gsem):
        pltpu.async_copy(table_hbm.at[idxv], rowsv, gsem)

    def wait_gather(idxv, rowsv, gsem):
        pltpu.make_async_copy(table_hbm.at[idxv], rowsv, gsem).wait()

    def wait_out(outv, osem):
        pltpu.make_async_copy(outv, nf_hbm.at[pl.ds(0, _CR)], osem).wait()

    def zrow(r, carry):
        for outv in (out0, out1):
            outv[r, pl.ds(_M, _L)] = jnp.zeros((_L,), jnp.float32)
            outv[r, pl.ds(_M + _L, _L)] = jnp.zeros((_L,), jnp.float32)
        return carry

    lax.fori_loop(0, _CR, zrow, 0)

    def compute(rowsv, wv, outv):
        def pt_body(pt, carry):
            def a_body(a, c2):
                wvk = [wv[pt, a, k, :] for k in range(_KS)]
                acc = [[jnp.zeros((_L,), jnp.float32) for _ in range(2)]
                       for _ in range(_KS)]
                for n in range(_NN):
                    r0 = rowsv[pt * _NN + n, pl.ds(a * _C_IN, _L)]
                    r1 = rowsv[pt * _NN + n, pl.ds(a * _C_IN + _L, _L)]
                    for k in range(_KS):
                        s = wvk[k][n]
                        acc[k][0] = acc[k][0] + s * r0
                        acc[k][1] = acc[k][1] + s * r1
                for k in range(_KS):
                    outv[pt * _A + a, pl.ds(k * _C_IN, _L)] = acc[k][0]
                    outv[pt * _A + a, pl.ds(k * _C_IN + _L, _L)] = acc[k][1]
                return c2

            lax.fori_loop(0, _A, a_body, 0, unroll=2)
            return carry

        lax.fori_loop(0, _CH, pt_body, 0)

    c0 = wid
    c1 = wid + _NW

    @pl.when(c0 < _NCHUNK)
    def _():
        pltpu.sync_copy(idx_hbm.at[pl.ds(c0 * _CH * _NN, _CH * _NN)], idx0)
        pltpu.sync_copy(w_hbm.at[pl.ds(c0 * _CH, _CH)], w0)
        start_gather(idx0, rows0, g0)

    @pl.when(c1 < _NCHUNK)
    def _():
        start_idx(c1, idx1, ic1)
        start_w(c1, w1, ws1)

    def loop(ch2, carry):
        cA = (2 * ch2) * _NW + wid
        cB = cA + _NW

        @pl.when(cA < _NCHUNK)
        def _():
            wait_gather(idx0, rows0, g0)

            @pl.when(cA + 2 * _NW < _NCHUNK)
            def _():
                start_idx(cA + 2 * _NW, idx0, ic0)

            @pl.when(cB < _NCHUNK)
            def _():
                wait_idx(idx1, ic1)
                start_gather(idx1, rows1, g1)

            @pl.when(ch2 >= 1)
            def _():
                wait_w(w0, ws0)
                wait_out(out0, o0)

            compute(rows0, w0, out0)
            pltpu.async_copy(out0, nf_hbm.at[pl.ds(cA * _CR, _CR)], o0)

            @pl.when(cA + 2 * _NW < _NCHUNK)
            def _():
                start_w(cA + 2 * _NW, w0, ws0)

        @pl.when(cB < _NCHUNK)
        def _():
            wait_gather(idx1, rows1, g1)

            @pl.when(cB + 2 * _NW < _NCHUNK)
            def _():
                start_idx(cB + 2 * _NW, idx1, ic1)

            @pl.when(cB + _NW < _NCHUNK)
            def _():
                wait_idx(idx0, ic0)
                start_gather(idx0, rows0, g0)

            wait_w(w1, ws1)

            @pl.when(ch2 >= 1)
            def _():
                wait_out(out1, o1)

            compute(rows1, w1, out1)
            pltpu.async_copy(out1, nf_hbm.at[pl.ds(cB * _CR, _CR)], o1)

            @pl.when(cB + 2 * _NW < _NCHUNK)
            def _():
                start_w(cB + 2 * _NW, w1, ws1)

        return carry

    lax.fori_loop(0, (_ITERS + 1) // 2, loop, 0)

    @pl.when(n_me >= 1)
    def _():
        wait_out(out0, o0)

    @pl.when(n_me >= 2)
    def _():
        wait_out(out1, o1)


@functools.cache
def _get_sc_nf():
    return pl.kernel(
        _sc_body,
        out_type=jax.ShapeDtypeStruct((_N * _A, _MP), jnp.float32),
        mesh=plsc.VectorSubcoreMesh(
            core_axis_name="c", subcore_axis_name="s",
            num_cores=_NC, num_subcores=_NS),
        scratch_types=[
            pltpu.VMEM((_CH * _NN,), jnp.int32),
            pltpu.VMEM((_CH * _NN,), jnp.int32),
            pltpu.VMEM((_CH * _NN, _D), jnp.float32),
            pltpu.VMEM((_CH * _NN, _D), jnp.float32),
            pltpu.VMEM((_CH, _A, _KS, _NN), jnp.float32),
            pltpu.VMEM((_CH, _A, _KS, _NN), jnp.float32),
            pltpu.VMEM((_CR, _MP), jnp.float32),
            pltpu.VMEM((_CR, _MP), jnp.float32),
            pltpu.SemaphoreType.DMA,
            pltpu.SemaphoreType.DMA,
            pltpu.SemaphoreType.DMA,
            pltpu.SemaphoreType.DMA,
            pltpu.SemaphoreType.DMA,
            pltpu.SemaphoreType.DMA,
            pltpu.SemaphoreType.DMA,
            pltpu.SemaphoreType.DMA,
        ],
    )


_RB = 1000
_G1 = (_N * _A) // _RB


def _tc_body(w2_ref, x_ref, o_ref, st_ref):
    ph = pl.program_id(0)
    i = pl.program_id(1)

    @pl.when(ph == 0)
    def _():
        y = lax.dot_general(x_ref[...], w2_ref[...], (((1,), (0,)), ((), ())),
                            preferred_element_type=jnp.float32)
        o_ref[pl.ds(i * _RB, _RB), :] = y

        @pl.when(i == 0)
        def _():
            st_ref[...] = jnp.zeros_like(st_ref)

        st_ref[0, :] += jnp.sum(y, axis=0)
        st_ref[1, :] += jnp.sum(y * y, axis=0)

    @pl.when(ph == 1)
    def _():
        cnt = float(_N * _A)
        mean = st_ref[0, :] / cnt
        var = st_ref[1, :] / cnt - mean * mean
        inv = lax.rsqrt(var + 1e-5)
        y = o_ref[pl.ds(i * _RB, _RB), :]
        o_ref[pl.ds(i * _RB, _RB), :] = jnp.maximum(
            (y - mean[None, :]) * inv[None, :], 0.0)


_tc_call = pl.pallas_call(
    _tc_body,
    grid=(2, _G1),
    in_specs=[
        pl.BlockSpec((_MP, _C_OUT), lambda ph, i: (0, 0)),
        pl.BlockSpec((_RB, _MP), lambda ph, i: (i * (1 - ph), 0)),
    ],
    out_specs=pl.BlockSpec((_N * _A, _C_OUT), lambda ph, i: (0, 0)),
    out_shape=jax.ShapeDtypeStruct((_N * _A, _C_OUT), jnp.float32),
    scratch_shapes=[pltpu.VMEM((2, _C_OUT), jnp.float32)],
    compiler_params=pltpu.CompilerParams(vmem_limit_bytes=100 * 1024 * 1024),
)


def kernel(xyz, feats, anchors, inter_idx, inter_w, W):
    table = feats[0].transpose(1, 2, 0).reshape(_N, _D)
    idx = inter_idx[0].reshape(_N * _NN).astype(jnp.int32)
    wgt = inter_w[0]
    x = _get_sc_nf()(table, idx, wgt)
    w2 = W.reshape(_C_OUT, _C_IN, _KS).transpose(2, 1, 0).reshape(_M, _C_OUT)
    w2p = jnp.pad(w2, ((0, _MP - _M), (0, 0)))
    out = _tc_call(w2p, x)
    return out.T.reshape(1, _C_OUT, _N, _A)


# speedup vs baseline: 34.5041x; 1.0008x over previous
import functools

import jax
import jax.numpy as jnp
from jax import lax
from jax.experimental import pallas as pl
from jax.experimental.pallas import tpu as pltpu
from jax.experimental.pallas import tpu_sc as plsc

_C_IN, _N, _A, _NN, _KS, _C_OUT = 32, 10000, 12, 16, 3, 64
_D = _A * _C_IN
_M = _KS * _C_IN
_MP = 128
_L = 16

_NC, _NS = 2, 16
_NW = _NC * _NS
_CH = 4
_CR = _CH * _A
_NCHUNK = _N // _CH
_ITERS = -(-_NCHUNK // _NW)


def _sc_body(table_hbm, idx_hbm, w_hbm, nf_hbm,
             idx0, idx1, rows0, rows1, w0, w1, out0, out1,
             g0, g1, o0, o1, ic0, ic1, ws0, ws1):
    wid = lax.axis_index("s") * _NC + lax.axis_index("c")
    n_me = (_NCHUNK - wid + _NW - 1) // _NW

    def start_idx(c, idxv, icsem):
        pltpu.async_copy(idx_hbm.at[pl.ds(c * _CH * _NN, _CH * _NN)], idxv,
                         icsem)

    def wait_idx(idxv, icsem):
        pltpu.make_async_copy(idx_hbm.at[pl.ds(0, _CH * _NN)], idxv,
                              icsem).wait()

    def start_w(c, wv, wsem):
        pltpu.async_copy(w_hbm.at[pl.ds(c * _CH, _CH)], wv, wsem)

    def wait_w(wv, wsem):
        pltpu.make_async_copy(w_hbm.at[pl.ds(0, _CH)], wv, wsem).wait()

    def start_gather(idxv, rowsv, gsem):
        pltpu.async_copy(table_hbm.at[idxv], rowsv, gsem)

    def wait_gather(idxv, rowsv, gsem):
        pltpu.make_async_copy(table_hbm.at[idxv], rowsv, gsem).wait()

    def wait_out(outv, osem):
        pltpu.make_async_copy(outv, nf_hbm.at[pl.ds(0, _CR)], osem).wait()

    def zrow(r, carry):
        for outv in (out0, out1):
            outv[r, pl.ds(_M, _L)] = jnp.zeros((_L,), jnp.float32)
            outv[r, pl.ds(_M + _L, _L)] = jnp.zeros((_L,), jnp.float32)
        return carry

    lax.fori_loop(0, _CR, zrow, 0)

    def compute(rowsv, wv, outv):
        def pt_body(pt, carry):
            def a_body(a, c2):
                wvk = [wv[pt, a, k, :] for k in range(_KS)]
                acc = [[jnp.zeros((_L,), jnp.float32) for _ in range(2)]
                       for _ in range(_KS)]
                for n in range(_NN):
                    r0 = rowsv[pt * _NN + n, pl.ds(a * _C_IN, _L)]
                    r1 = rowsv[pt * _NN + n, pl.ds(a * _C_IN + _L, _L)]
                    for k in range(_KS):
                        s = wvk[k][n]
                        acc[k][0] = acc[k][0] + s * r0
                        acc[k][1] = acc[k][1] + s * r1
                for k in range(_KS):
                    outv[pt * _A + a, pl.ds(k * _C_IN, _L)] = acc[k][0]
                    outv[pt * _A + a, pl.ds(k * _C_IN + _L, _L)] = acc[k][1]
                return c2

            lax.fori_loop(0, _A, a_body, 0, unroll=4)
            return carry

        lax.fori_loop(0, _CH, pt_body, 0)

    c0 = wid
    c1 = wid + _NW

    @pl.when(c0 < _NCHUNK)
    def _():
        pltpu.sync_copy(idx_hbm.at[pl.ds(c0 * _CH * _NN, _CH * _NN)], idx0)
        pltpu.sync_copy(w_hbm.at[pl.ds(c0 * _CH, _CH)], w0)
        start_gather(idx0, rows0, g0)

    @pl.when(c1 < _NCHUNK)
    def _():
        start_idx(c1, idx1, ic1)
        start_w(c1, w1, ws1)

    def loop(ch2, carry):
        cA = (2 * ch2) * _NW + wid
        cB = cA + _NW

        @pl.when(cA < _NCHUNK)
        def _():
            wait_gather(idx0, rows0, g0)

            @pl.when(cA + 2 * _NW < _NCHUNK)
            def _():
                start_idx(cA + 2 * _NW, idx0, ic0)

            @pl.when(cB < _NCHUNK)
            def _():
                wait_idx(idx1, ic1)
                start_gather(idx1, rows1, g1)

            @pl.when(ch2 >= 1)
            def _():
                wait_w(w0, ws0)
                wait_out(out0, o0)

            compute(rows0, w0, out0)
            pltpu.async_copy(out0, nf_hbm.at[pl.ds(cA * _CR, _CR)], o0)

            @pl.when(cA + 2 * _NW < _NCHUNK)
            def _():
                start_w(cA + 2 * _NW, w0, ws0)

        @pl.when(cB < _NCHUNK)
        def _():
            wait_gather(idx1, rows1, g1)

            @pl.when(cB + 2 * _NW < _NCHUNK)
            def _():
                start_idx(cB + 2 * _NW, idx1, ic1)

            @pl.when(cB + _NW < _NCHUNK)
            def _():
                wait_idx(idx0, ic0)
                start_gather(idx0, rows0, g0)

            wait_w(w1, ws1)

            @pl.when(ch2 >= 1)
            def _():
                wait_out(out1, o1)

            compute(rows1, w1, out1)
            pltpu.async_copy(out1, nf_hbm.at[pl.ds(cB * _CR, _CR)], o1)

            @pl.when(cB + 2 * _NW < _NCHUNK)
            def _():
                start_w(cB + 2 * _NW, w1, ws1)

        return carry

    lax.fori_loop(0, (_ITERS + 1) // 2, loop, 0)

    @pl.when(n_me >= 1)
    def _():
        wait_out(out0, o0)

    @pl.when(n_me >= 2)
    def _():
        wait_out(out1, o1)


@functools.cache
def _get_sc_nf():
    return pl.kernel(
        _sc_body,
        out_type=jax.ShapeDtypeStruct((_N * _A, _MP), jnp.float32),
        mesh=plsc.VectorSubcoreMesh(
            core_axis_name="c", subcore_axis_name="s",
            num_cores=_NC, num_subcores=_NS),
        scratch_types=[
            pltpu.VMEM((_CH * _NN,), jnp.int32),
            pltpu.VMEM((_CH * _NN,), jnp.int32),
            pltpu.VMEM((_CH * _NN, _D), jnp.float32),
            pltpu.VMEM((_CH * _NN, _D), jnp.float32),
            pltpu.VMEM((_CH, _A, _KS, _NN), jnp.float32),
            pltpu.VMEM((_CH, _A, _KS, _NN), jnp.float32),
            pltpu.VMEM((_CR, _MP), jnp.float32),
            pltpu.VMEM((_CR, _MP), jnp.float32),
            pltpu.SemaphoreType.DMA,
            pltpu.SemaphoreType.DMA,
            pltpu.SemaphoreType.DMA,
            pltpu.SemaphoreType.DMA,
            pltpu.SemaphoreType.DMA,
            pltpu.SemaphoreType.DMA,
            pltpu.SemaphoreType.DMA,
            pltpu.SemaphoreType.DMA,
        ],
    )


_RB = 1000
_G1 = (_N * _A) // _RB


def _tc_body(w2_ref, x_ref, o_ref, st_ref):
    ph = pl.program_id(0)
    i = pl.program_id(1)

    @pl.when(ph == 0)
    def _():
        y = lax.dot_general(x_ref[...].astype(jnp.bfloat16),
                            w2_ref[...].astype(jnp.bfloat16),
                            (((1,), (0,)), ((), ())),
                            preferred_element_type=jnp.float32)
        o_ref[pl.ds(i * _RB, _RB), :] = y

        @pl.when(i == 0)
        def _():
            st_ref[...] = jnp.zeros_like(st_ref)

        st_ref[0, :] += jnp.sum(y, axis=0)
        st_ref[1, :] += jnp.sum(y * y, axis=0)

    @pl.when(ph == 1)
    def _():
        cnt = float(_N * _A)
        mean = st_ref[0, :] / cnt
        var = st_ref[1, :] / cnt - mean * mean
        inv = lax.rsqrt(var + 1e-5)
        y = o_ref[pl.ds(i * _RB, _RB), :]
        o_ref[pl.ds(i * _RB, _RB), :] = jnp.maximum(
            (y - mean[None, :]) * inv[None, :], 0.0)


_tc_call = pl.pallas_call(
    _tc_body,
    grid=(2, _G1),
    in_specs=[
        pl.BlockSpec((_MP, _C_OUT), lambda ph, i: (0, 0)),
        pl.BlockSpec((_RB, _MP), lambda ph, i: (i * (1 - ph), 0)),
    ],
    out_specs=pl.BlockSpec((_N * _A, _C_OUT), lambda ph, i: (0, 0)),
    out_shape=jax.ShapeDtypeStruct((_N * _A, _C_OUT), jnp.float32),
    scratch_shapes=[pltpu.VMEM((2, _C_OUT), jnp.float32)],
    compiler_params=pltpu.CompilerParams(vmem_limit_bytes=100 * 1024 * 1024),
)


def kernel(xyz, feats, anchors, inter_idx, inter_w, W):
    table = feats[0].transpose(1, 2, 0).reshape(_N, _D)
    idx = inter_idx[0].reshape(_N * _NN).astype(jnp.int32)
    wgt = inter_w[0]
    x = _get_sc_nf()(table, idx, wgt)
    w2 = W.reshape(_C_OUT, _C_IN, _KS).transpose(2, 1, 0).reshape(_M, _C_OUT)
    w2p = jnp.pad(w2, ((0, _MP - _M), (0, 0)))
    out = _tc_call(w2p, x)
    return out.T.reshape(1, _C_OUT, _N, _A)


# speedup vs baseline: 36.9099x; 1.0697x over previous
import functools

import jax
import jax.numpy as jnp
from jax import lax
from jax.experimental import pallas as pl
from jax.experimental.pallas import tpu as pltpu
from jax.experimental.pallas import tpu_sc as plsc

_C_IN, _N, _A, _NN, _KS, _C_OUT = 32, 10000, 12, 16, 3, 64
_D = _A * _C_IN
_M = _KS * _C_IN
_MP = 128
_L = 16

_NC, _NS = 2, 16
_NW = _NC * _NS
_CH = 4
_CR = _CH * _A
_NCHUNK = _N // _CH
_ITERS = -(-_NCHUNK // _NW)


def _sc_body(table_hbm, idx_hbm, w_hbm, nf_hbm,
             idx0, idx1, rows0, rows1, w0, w1, out0, out1,
             g0, g1, o0, o1, ic0, ic1, ws0, ws1):
    wid = lax.axis_index("s") * _NC + lax.axis_index("c")
    n_me = (_NCHUNK - wid + _NW - 1) // _NW

    def start_idx(c, idxv, icsem):
        pltpu.async_copy(idx_hbm.at[pl.ds(c * _CH * _NN, _CH * _NN)], idxv,
                         icsem)

    def wait_idx(idxv, icsem):
        pltpu.make_async_copy(idx_hbm.at[pl.ds(0, _CH * _NN)], idxv,
                              icsem).wait()

    def start_w(c, wv, wsem):
        pltpu.async_copy(w_hbm.at[pl.ds(c * _CH, _CH)], wv, wsem)

    def wait_w(wv, wsem):
        pltpu.make_async_copy(w_hbm.at[pl.ds(0, _CH)], wv, wsem).wait()

    def start_gather(idxv, rowsv, gsem):
        pltpu.async_copy(table_hbm.at[idxv], rowsv, gsem)

    def wait_gather(idxv, rowsv, gsem):
        pltpu.make_async_copy(table_hbm.at[idxv], rowsv, gsem).wait()

    def wait_out(outv, osem):
        pltpu.make_async_copy(outv, nf_hbm.at[pl.ds(0, _CR)], osem).wait()

    def zrow(r, carry):
        for outv in (out0, out1):
            outv[r, pl.ds(_M, _L)] = jnp.zeros((_L,), jnp.float32)
            outv[r, pl.ds(_M + _L, _L)] = jnp.zeros((_L,), jnp.float32)
        return carry

    lax.fori_loop(0, _CR, zrow, 0)

    def compute(rowsv, wv, outv):
        def pt_body(pt, carry):
            def a_body(a, c2):
                wvk = [wv[pt, a, k, :] for k in range(_KS)]
                acc = [[jnp.zeros((_L,), jnp.float32) for _ in range(2)]
                       for _ in range(_KS)]
                for n in range(_NN):
                    r0 = rowsv[pt * _NN + n, pl.ds(a * _C_IN, _L)]
                    r1 = rowsv[pt * _NN + n, pl.ds(a * _C_IN + _L, _L)]
                    for k in range(_KS):
                        s = wvk[k][n]
                        acc[k][0] = acc[k][0] + s * r0
                        acc[k][1] = acc[k][1] + s * r1
                for k in range(_KS):
                    outv[pt * _A + a, pl.ds(k * _C_IN, _L)] = acc[k][0]
                    outv[pt * _A + a, pl.ds(k * _C_IN + _L, _L)] = acc[k][1]
                return c2

            lax.fori_loop(0, _A, a_body, 0, unroll=2)
            return carry

        lax.fori_loop(0, _CH, pt_body, 0)

    c0 = wid
    c1 = wid + _NW

    @pl.when(c0 < _NCHUNK)
    def _():
        pltpu.sync_copy(idx_hbm.at[pl.ds(c0 * _CH * _NN, _CH * _NN)], idx0)
        pltpu.sync_copy(w_hbm.at[pl.ds(c0 * _CH, _CH)], w0)
        start_gather(idx0, rows0, g0)

    @pl.when(c1 < _NCHUNK)
    def _():
        start_idx(c1, idx1, ic1)
        start_w(c1, w1, ws1)

    def loop(ch2, carry):
        cA = (2 * ch2) * _NW + wid
        cB = cA + _NW

        @pl.when(cA < _NCHUNK)
        def _():
            wait_gather(idx0, rows0, g0)

            @pl.when(cA + 2 * _NW < _NCHUNK)
            def _():
                start_idx(cA + 2 * _NW, idx0, ic0)

            @pl.when(cB < _NCHUNK)
            def _():
                wait_idx(idx1, ic1)
                start_gather(idx1, rows1, g1)

            @pl.when(ch2 >= 1)
            def _():
                wait_w(w0, ws0)
                wait_out(out0, o0)

            compute(rows0, w0, out0)
            pltpu.async_copy(out0, nf_hbm.at[pl.ds(cA * _CR, _CR)], o0)

            @pl.when(cA + 2 * _NW < _NCHUNK)
            def _():
                start_w(cA + 2 * _NW, w0, ws0)

        @pl.when(cB < _NCHUNK)
        def _():
            wait_gather(idx1, rows1, g1)

            @pl.when(cB + 2 * _NW < _NCHUNK)
            def _():
                start_idx(cB + 2 * _NW, idx1, ic1)

            @pl.when(cB + _NW < _NCHUNK)
            def _():
                wait_idx(idx0, ic0)
                start_gather(idx0, rows0, g0)

            wait_w(w1, ws1)

            @pl.when(ch2 >= 1)
            def _():
                wait_out(out1, o1)

            compute(rows1, w1, out1)
            pltpu.async_copy(out1, nf_hbm.at[pl.ds(cB * _CR, _CR)], o1)

            @pl.when(cB + 2 * _NW < _NCHUNK)
            def _():
                start_w(cB + 2 * _NW, w1, ws1)

        return carry

    lax.fori_loop(0, (_ITERS + 1) // 2, loop, 0)

    @pl.when(n_me >= 1)
    def _():
        wait_out(out0, o0)

    @pl.when(n_me >= 2)
    def _():
        wait_out(out1, o1)


@functools.cache
def _get_sc_nf():
    return pl.kernel(
        _sc_body,
        out_type=jax.ShapeDtypeStruct((_N * _A, _MP), jnp.float32),
        mesh=plsc.VectorSubcoreMesh(
            core_axis_name="c", subcore_axis_name="s",
            num_cores=_NC, num_subcores=_NS),
        scratch_types=[
            pltpu.VMEM((_CH * _NN,), jnp.int32),
            pltpu.VMEM((_CH * _NN,), jnp.int32),
            pltpu.VMEM((_CH * _NN, _D), jnp.float32),
            pltpu.VMEM((_CH * _NN, _D), jnp.float32),
            pltpu.VMEM((_CH, _A, _KS, _NN), jnp.float32),
            pltpu.VMEM((_CH, _A, _KS, _NN), jnp.float32),
            pltpu.VMEM((_CR, _MP), jnp.float32),
            pltpu.VMEM((_CR, _MP), jnp.float32),
            pltpu.SemaphoreType.DMA,
            pltpu.SemaphoreType.DMA,
            pltpu.SemaphoreType.DMA,
            pltpu.SemaphoreType.DMA,
            pltpu.SemaphoreType.DMA,
            pltpu.SemaphoreType.DMA,
            pltpu.SemaphoreType.DMA,
            pltpu.SemaphoreType.DMA,
        ],
    )


_RB = 2000
_G1 = (_N * _A) // _RB


def _tc_body(w2_ref, x_ref, o_ref, st_ref):
    ph = pl.program_id(0)
    i = pl.program_id(1)

    @pl.when(ph == 0)
    def _():
        y = lax.dot_general(x_ref[...].astype(jnp.bfloat16),
                            w2_ref[...].astype(jnp.bfloat16),
                            (((1,), (0,)), ((), ())),
                            preferred_element_type=jnp.float32)
        o_ref[pl.ds(i * _RB, _RB), :] = y

        @pl.when(i == 0)
        def _():
            st_ref[...] = jnp.zeros_like(st_ref)

        st_ref[0, :] += jnp.sum(y, axis=0)
        st_ref[1, :] += jnp.sum(y * y, axis=0)

    @pl.when(ph == 1)
    def _():
        cnt = float(_N * _A)
        mean = st_ref[0, :] / cnt
        var = st_ref[1, :] / cnt - mean * mean
        inv = lax.rsqrt(var + 1e-5)
        y = o_ref[pl.ds(i * _RB, _RB), :]
        o_ref[pl.ds(i * _RB, _RB), :] = jnp.maximum(
            (y - mean[None, :]) * inv[None, :], 0.0)


_tc_call = pl.pallas_call(
    _tc_body,
    grid=(2, _G1),
    in_specs=[
        pl.BlockSpec((_MP, _C_OUT), lambda ph, i: (0, 0)),
        pl.BlockSpec((_RB, _MP), lambda ph, i: (i * (1 - ph), 0)),
    ],
    out_specs=pl.BlockSpec((_N * _A, _C_OUT), lambda ph, i: (0, 0)),
    out_shape=jax.ShapeDtypeStruct((_N * _A, _C_OUT), jnp.float32),
    scratch_shapes=[pltpu.VMEM((2, _C_OUT), jnp.float32)],
    compiler_params=pltpu.CompilerParams(vmem_limit_bytes=100 * 1024 * 1024),
)


def kernel(xyz, feats, anchors, inter_idx, inter_w, W):
    table = feats[0].transpose(1, 2, 0).reshape(_N, _D)
    idx = inter_idx[0].reshape(_N * _NN).astype(jnp.int32)
    wgt = inter_w[0]
    x = _get_sc_nf()(table, idx, wgt)
    w2 = W.reshape(_C_OUT, _C_IN, _KS).transpose(2, 1, 0).reshape(_M, _C_OUT)
    w2p = jnp.pad(w2, ((0, _MP - _M), (0, 0)))
    out = _tc_call(w2p, x)
    return out.reshape(_N, _A, _C_OUT).transpose(2, 0, 1)[None]
